# baseline (speedup 1.0000x reference)
import jax
import jax.numpy as jnp
from jax.experimental import pallas as pl
from jax.experimental.pallas import tpu as pltpu

BT = 2048


def _router_block(x_ref, w_ref, b_ref, logits_ref, probs_ref):
    logits = jnp.dot(x_ref[...], w_ref[...], preferred_element_type=jnp.float32)
    logits = logits + b_ref[...]
    logits_ref[...] = logits
    na = logits.shape[-1] // 2
    ps = []
    for s in range(2):
        l = logits[:, s * na : (s + 1) * na]
        m = jnp.max(l, axis=-1, keepdims=True)
        e = jnp.exp(l - m)
        ps.append(e / jnp.sum(e, axis=-1, keepdims=True))
    probs_ref[...] = jnp.concatenate(ps, axis=1)


def kernel(x, W, b):
    tokens, d = x.shape
    na = W.shape[1]
    x2 = x.reshape(tokens // 2, 2 * d)
    z = jnp.zeros_like(W)
    W2 = jnp.concatenate(
        [jnp.concatenate([W, z], axis=1), jnp.concatenate([z, W], axis=1)], axis=0
    )
    b2 = jnp.concatenate([b, b]).reshape(1, 2 * na)
    rows = tokens // 2
    out_shape = jax.ShapeDtypeStruct((rows, 2 * na), jnp.float32)
    logits2, probs2 = pl.pallas_call(
        _router_block,
        grid=(rows // BT,),
        in_specs=[
            pl.BlockSpec((BT, 2 * d), lambda i: (i, 0)),
            pl.BlockSpec((2 * d, 2 * na), lambda i: (0, 0)),
            pl.BlockSpec((1, 2 * na), lambda i: (0, 0)),
        ],
        out_specs=[
            pl.BlockSpec((BT, 2 * na), lambda i: (i, 0)),
            pl.BlockSpec((BT, 2 * na), lambda i: (i, 0)),
        ],
        out_shape=[out_shape, out_shape],
        compiler_params=pltpu.CompilerParams(
            dimension_semantics=(pltpu.PARALLEL,),
            vmem_limit_bytes=100 * 1024 * 1024,
        ),
    )(x2, W2, b2)
    return (logits2.reshape(tokens, na), probs2.reshape(tokens, na))


# speedup vs baseline: 1.0414x; 1.0414x over previous
import jax
import jax.numpy as jnp
from jax.experimental import pallas as pl
from jax.experimental.pallas import tpu as pltpu

BT = 2048


def _router_block(x_ref, w_ref, b_ref, bd_ref, logits_ref, probs_ref):
    xb = x_ref[...]
    d = w_ref.shape[0]
    w = w_ref[...]
    le = jnp.dot(xb[:, :d], w, preferred_element_type=jnp.float32)
    lo = jnp.dot(xb[:, d:], w, preferred_element_type=jnp.float32)
    l = jnp.concatenate([le, lo], axis=1) + b_ref[...]
    logits_ref[...] = l
    m = jnp.max(l, axis=-1, keepdims=True)
    e = jnp.exp(l - m)
    s = jnp.dot(e, bd_ref[...], preferred_element_type=jnp.float32)
    probs_ref[...] = e / s


def kernel(x, W, b):
    tokens, d = x.shape
    na = W.shape[1]
    x2 = x.reshape(tokens // 2, 2 * d)
    b2 = jnp.concatenate([b, b]).reshape(1, 2 * na)
    g = jnp.arange(2 * na) // na
    bd = (g[:, None] == g[None, :]).astype(jnp.float32)
    rows = tokens // 2
    out_shape = jax.ShapeDtypeStruct((rows, 2 * na), jnp.float32)
    logits2, probs2 = pl.pallas_call(
        _router_block,
        grid=(rows // BT,),
        in_specs=[
            pl.BlockSpec((BT, 2 * d), lambda i: (i, 0)),
            pl.BlockSpec((d, na), lambda i: (0, 0)),
            pl.BlockSpec((1, 2 * na), lambda i: (0, 0)),
            pl.BlockSpec((2 * na, 2 * na), lambda i: (0, 0)),
        ],
        out_specs=[
            pl.BlockSpec((BT, 2 * na), lambda i: (i, 0)),
            pl.BlockSpec((BT, 2 * na), lambda i: (i, 0)),
        ],
        out_shape=[out_shape, out_shape],
        compiler_params=pltpu.CompilerParams(
            dimension_semantics=(pltpu.PARALLEL,),
            vmem_limit_bytes=100 * 1024 * 1024,
        ),
    )(x2, W, b2, bd)
    return (logits2.reshape(tokens, na), probs2.reshape(tokens, na))


# speedup vs baseline: 3.1211x; 2.9969x over previous
import jax
import jax.numpy as jnp
from jax.experimental import pallas as pl
from jax.experimental.pallas import tpu as pltpu

CH = 1024
NBUF = 6


def _router_kernel(x_hbm, w_ref, b_ref, logits_hbm, probs_hbm,
                   buf, logits_v, probs_v, insems, outsem):
    tokens = x_hbm.shape[0]
    nchunks = tokens // CH

    def copy_in(t, slot):
        return pltpu.make_async_copy(
            x_hbm.at[pl.ds(t * CH, CH), :],
            buf.at[slot],
            insems.at[slot],
        )

    for k in range(NBUF):
        copy_in(k, k).start()

    def body(t, carry):
        slot = jax.lax.rem(t, NBUF)
        copy_in(t, slot).wait()
        logits = jnp.dot(buf[slot], w_ref[...],
                         preferred_element_type=jnp.float32)
        logits = logits + b_ref[...]
        rows = pl.ds(t * CH, CH)
        logits_v[rows, :] = logits
        m = jnp.max(logits, axis=-1, keepdims=True)
        e = jnp.exp(logits - m)
        probs_v[rows, :] = e / jnp.sum(e, axis=-1, keepdims=True)
        pltpu.make_async_copy(
            logits_v.at[rows, :], logits_hbm.at[rows, :], outsem.at[0]
        ).start()
        pltpu.make_async_copy(
            probs_v.at[rows, :], probs_hbm.at[rows, :], outsem.at[1]
        ).start()

        @pl.when(t + NBUF < nchunks)
        def _prefetch():
            copy_in(t + NBUF, jax.lax.rem(t + NBUF, NBUF)).start()

        return carry

    jax.lax.fori_loop(0, nchunks, body, 0)

    for t in range(nchunks):
        rows = pl.ds(t * CH, CH)
        pltpu.make_async_copy(
            logits_v.at[rows, :], logits_hbm.at[rows, :], outsem.at[0]
        ).wait()
        pltpu.make_async_copy(
            probs_v.at[rows, :], probs_hbm.at[rows, :], outsem.at[1]
        ).wait()


def kernel(x, W, b):
    tokens, d = x.shape
    na = W.shape[1]
    b2 = b.reshape(1, na)
    out_shape = jax.ShapeDtypeStruct((tokens, na), jnp.float32)
    logits, probs = pl.pallas_call(
        _router_kernel,
        grid=(1,),
        in_specs=[
            pl.BlockSpec(memory_space=pltpu.HBM),
            pl.BlockSpec((d, na), lambda i: (0, 0)),
            pl.BlockSpec((1, na), lambda i: (0, 0)),
        ],
        out_specs=[
            pl.BlockSpec(memory_space=pltpu.HBM),
            pl.BlockSpec(memory_space=pltpu.HBM),
        ],
        out_shape=[out_shape, out_shape],
        scratch_shapes=[
            pltpu.VMEM((NBUF, CH, d), jnp.float32),
            pltpu.VMEM((tokens, na), jnp.float32),
            pltpu.VMEM((tokens, na), jnp.float32),
            pltpu.SemaphoreType.DMA((NBUF,)),
            pltpu.SemaphoreType.DMA((2,)),
        ],
        compiler_params=pltpu.CompilerParams(
            dimension_semantics=(pltpu.ARBITRARY,),
            vmem_limit_bytes=100 * 1024 * 1024,
        ),
    )(x, W, b2)
    return (logits, probs)
